# baseline (speedup 1.0000x reference)
import functools

import jax
import jax.numpy as jnp
from jax import lax
from jax.experimental import pallas as pl
from jax.experimental.pallas import tpu as pltpu
from jax.experimental.pallas import tpu_sc as plsc

TAB_ROWS = 101
D = 64
B = 4096 * 200
IR_W = 128
NC, NS = 2, 16
NW = NC * NS
B_PER_W = B // NW
IR_PER_W = B_PER_W // IR_W
CHUNK_IR = 4
CHUNK = CHUNK_IR * IR_W
N_CHUNKS = B_PER_W // CHUNK


def _sc_gather(z2d, table):
    mesh = plsc.VectorSubcoreMesh(core_axis_name="c", subcore_axis_name="s")

    @functools.partial(
        pl.kernel,
        mesh=mesh,
        out_type=jax.ShapeDtypeStruct((B, D), jnp.float32),
        scratch_types=[
            pltpu.VMEM_SHARED((TAB_ROWS, D), jnp.float32),
            pltpu.VMEM((TAB_ROWS, D), jnp.float32),
            pltpu.VMEM((IR_PER_W, IR_W), jnp.int32),
            pltpu.VMEM((2, CHUNK, D), jnp.float32),
            pltpu.SemaphoreType.DMA,
        ],
        compiler_params=pltpu.CompilerParams(use_tc_tiling_on_sc=False),
    )
    def k(z_hbm, table_hbm, out_hbm, table_sh, table_v, idx_v, rows_v, gsem):
        sid = lax.axis_index("s")
        wid = sid * NC + lax.axis_index("c")
        ir_base = wid * IR_PER_W
        row_base = wid * B_PER_W

        @pl.when(sid == 0)
        def _():
            pltpu.sync_copy(table_hbm, table_v)
            pltpu.sync_copy(table_v, table_sh)

        pltpu.sync_copy(z_hbm.at[pl.ds(ir_base, IR_PER_W)], idx_v)
        plsc.subcore_barrier()

        def fire_gathers(c, buf):
            for j in range(CHUNK_IR):
                pltpu.async_copy(
                    table_sh.at[idx_v.at[c * CHUNK_IR + j]],
                    rows_v.at[buf].at[pl.ds(j * IR_W, IR_W)],
                    gsem,
                )

        def wait_gathers(buf):
            for j in range(CHUNK_IR):
                pltpu.make_async_copy(
                    table_sh.at[idx_v.at[j]],
                    rows_v.at[buf].at[pl.ds(j * IR_W, IR_W)],
                    gsem,
                ).wait()

        def copy_out(c, buf):
            pltpu.sync_copy(rows_v.at[buf],
                            out_hbm.at[pl.ds(row_base + c * CHUNK, CHUNK)])

        fire_gathers(0, 0)

        def body(g, carry):
            for b in range(2):
                c = 2 * g + b
                fire_gathers(c + 1, 1 - b)
                wait_gathers(b)
                copy_out(c, b)
            return carry

        lax.fori_loop(0, (N_CHUNKS - 2) // 2, body, None)

        fire_gathers(N_CHUNKS - 1, 1)
        wait_gathers(0)
        copy_out(N_CHUNKS - 2, 0)
        wait_gathers(1)
        copy_out(N_CHUNKS - 1, 1)

    return k(z2d, table)


def kernel(z, table):
    zi = z.astype(jnp.int32)
    z2d = (zi & 127).reshape(B // IR_W, IR_W)
    out = _sc_gather(z2d, table)
    one = jnp.where(zi[0, 0] >= 0, jnp.float32(1), jnp.float32(2))
    return out.reshape(z.shape[0], z.shape[1], D) * one


# speedup vs baseline: 2.3664x; 2.3664x over previous
import functools

import jax
import jax.numpy as jnp
from jax import lax
from jax.experimental import pallas as pl
from jax.experimental.pallas import tpu as pltpu
from jax.experimental.pallas import tpu_sc as plsc

TAB_ROWS = 101
D = 64
DP = 128
B = 4096 * 200
IR_W = 128
NC, NS = 2, 16
NW = NC * NS
B_PER_W = B // NW
IR_PER_W = B_PER_W // IR_W
CHUNK_IR = 2
CHUNK = CHUNK_IR * IR_W
N_CHUNKS = B_PER_W // CHUNK


def _sc_gather(z2d, table):
    mesh = plsc.VectorSubcoreMesh(core_axis_name="c", subcore_axis_name="s")

    @functools.partial(
        pl.kernel,
        mesh=mesh,
        out_type=jax.ShapeDtypeStruct((B, DP), jnp.float32),
        scratch_types=[
            pltpu.VMEM_SHARED((TAB_ROWS, DP), jnp.float32),
            pltpu.VMEM((TAB_ROWS, DP), jnp.float32),
            pltpu.VMEM((IR_PER_W, IR_W), jnp.int32),
            pltpu.VMEM((2, CHUNK, DP), jnp.float32),
            pltpu.SemaphoreType.DMA,
        ],
        compiler_params=pltpu.CompilerParams(use_tc_tiling_on_sc=False),
    )
    def k(z_hbm, table_hbm, out_hbm, table_sh, table_v, idx_v, rows_v, gsem):
        sid = lax.axis_index("s")
        wid = sid * NC + lax.axis_index("c")
        ir_base = wid * IR_PER_W
        row_base = wid * B_PER_W

        @pl.when(sid == 0)
        def _():
            pltpu.sync_copy(table_hbm, table_v)
            pltpu.sync_copy(table_v, table_sh)

        pltpu.sync_copy(z_hbm.at[pl.ds(ir_base, IR_PER_W)], idx_v)
        plsc.subcore_barrier()

        def fire_gathers(c, buf):
            for j in range(CHUNK_IR):
                pltpu.async_copy(
                    table_sh.at[idx_v.at[c * CHUNK_IR + j]],
                    rows_v.at[buf].at[pl.ds(j * IR_W, IR_W)],
                    gsem,
                )

        def wait_gathers(buf):
            for j in range(CHUNK_IR):
                pltpu.make_async_copy(
                    table_sh.at[idx_v.at[j]],
                    rows_v.at[buf].at[pl.ds(j * IR_W, IR_W)],
                    gsem,
                ).wait()

        def copy_out(c, buf):
            pltpu.sync_copy(rows_v.at[buf],
                            out_hbm.at[pl.ds(row_base + c * CHUNK, CHUNK)])

        fire_gathers(0, 0)

        def body(g, carry):
            for b in range(2):
                c = 2 * g + b
                fire_gathers(c + 1, 1 - b)
                wait_gathers(b)
                copy_out(c, b)
            return carry

        lax.fori_loop(0, (N_CHUNKS - 2) // 2, body, None)

        fire_gathers(N_CHUNKS - 1, 1)
        wait_gathers(0)
        copy_out(N_CHUNKS - 2, 0)
        wait_gathers(1)
        copy_out(N_CHUNKS - 1, 1)

    return k(z2d, table)


def kernel(z, table):
    z2d = (z.astype(jnp.int32) & 127).reshape(B // IR_W, IR_W)
    table_pad = jnp.pad(table, ((0, 0), (0, DP - D)))
    out = _sc_gather(z2d, table_pad)
    return out[:, :D].reshape(z.shape[0], z.shape[1], D)


# speedup vs baseline: 2.5473x; 1.0765x over previous
import functools

import jax
import jax.numpy as jnp
from jax import lax
from jax.experimental import pallas as pl
from jax.experimental.pallas import tpu as pltpu
from jax.experimental.pallas import tpu_sc as plsc

TAB_ROWS = 101
D = 64
DP = 128
B = 4096 * 200
IR_W = 128
NC, NS = 2, 16
NW = NC * NS
B_PER_W = B // NW
IR_PER_W = B_PER_W // IR_W
CHUNK_IR = 2
CHUNK = CHUNK_IR * IR_W
N_CHUNKS = B_PER_W // CHUNK


def _sc_gather(z2d, table):
    mesh = plsc.VectorSubcoreMesh(core_axis_name="c", subcore_axis_name="s")

    @functools.partial(
        pl.kernel,
        mesh=mesh,
        out_type=jax.ShapeDtypeStruct((B, D), jnp.float32),
        scratch_types=[
            pltpu.VMEM_SHARED((TAB_ROWS, D), jnp.float32),
            pltpu.VMEM((TAB_ROWS, D), jnp.float32),
            pltpu.VMEM((IR_PER_W, IR_W), jnp.int32),
            pltpu.VMEM((2, CHUNK, D), jnp.float32),
            pltpu.SemaphoreType.DMA,
        ],
        compiler_params=pltpu.CompilerParams(use_tc_tiling_on_sc=True),
    )
    def k(z_hbm, table_hbm, out_hbm, table_sh, table_v, idx_v, rows_v, gsem):
        sid = lax.axis_index("s")
        wid = sid * NC + lax.axis_index("c")
        ir_base = wid * IR_PER_W
        row_base = wid * B_PER_W

        @pl.when(sid == 0)
        def _():
            pltpu.sync_copy(table_hbm, table_v)
            pltpu.sync_copy(table_v, table_sh)

        pltpu.sync_copy(z_hbm.at[pl.ds(ir_base, IR_PER_W)], idx_v)
        plsc.subcore_barrier()

        def fire_gathers(c, buf):
            for j in range(CHUNK_IR):
                pltpu.async_copy(
                    table_sh.at[idx_v.at[c * CHUNK_IR + j]],
                    rows_v.at[buf].at[pl.ds(j * IR_W, IR_W)],
                    gsem,
                )

        def wait_gathers(buf):
            for j in range(CHUNK_IR):
                pltpu.make_async_copy(
                    table_sh.at[idx_v.at[j]],
                    rows_v.at[buf].at[pl.ds(j * IR_W, IR_W)],
                    gsem,
                ).wait()

        def copy_out(c, buf):
            pltpu.sync_copy(rows_v.at[buf],
                            out_hbm.at[pl.ds(row_base + c * CHUNK, CHUNK)])

        fire_gathers(0, 0)

        def body(g, carry):
            for b in range(2):
                c = 2 * g + b
                fire_gathers(c + 1, 1 - b)
                wait_gathers(b)
                copy_out(c, b)
            return carry

        lax.fori_loop(0, (N_CHUNKS - 2) // 2, body, None)

        fire_gathers(N_CHUNKS - 1, 1)
        wait_gathers(0)
        copy_out(N_CHUNKS - 2, 0)
        wait_gathers(1)
        copy_out(N_CHUNKS - 1, 1)

    return k(z2d, table)


def kernel(z, table):
    z2d = (z.astype(jnp.int32) & 127).reshape(B // IR_W, IR_W)
    out = _sc_gather(z2d, table)
    return out.reshape(z.shape[0], z.shape[1], D)
